# baseline (speedup 1.0000x reference)
import functools

import jax
import jax.numpy as jnp
from jax import lax
from jax.experimental import pallas as pl
from jax.experimental.pallas import tpu as pltpu
from jax.experimental.pallas import tpu_sc as plsc

VOCAB = 100000
DIM = 128
BATCH = 16384

NUM_CORES = 2
NUM_SUBCORES = 16
NW = NUM_CORES * NUM_SUBCORES
BPW = BATCH // NW
CH = 128
NCH = BPW // CH
NBUF = 3
LANES = 16


def _lane_permute(x, idx):
    dnums = lax.GatherDimensionNumbers(
        offset_dims=(), collapsed_slice_dims=(0,), start_index_map=(0,))
    return lax.gather(
        x, idx[:, None], dnums, slice_sizes=(1,),
        mode=lax.GatherScatterMode.PROMISE_IN_BOUNDS)


def _dot_chunk(a_v, b_v, out_v, slot, out0, n):
    lanes = lax.broadcasted_iota(jnp.int32, (LANES,), 0)
    perms = [lanes ^ step for step in (8, 4, 2, 1)]
    lane0 = lanes == 0

    @plsc.parallel_loop(0, n)
    def _(row):
        acc = None
        for k in range(DIM // LANES):
            av = a_v[slot, row, pl.ds(k * LANES, LANES)]
            bv = b_v[slot, row, pl.ds(k * LANES, LANES)]
            prod = av * bv
            acc = prod if acc is None else acc + prod
        for perm in perms:
            acc = acc + _lane_permute(acc, perm)
        plsc.store_scatter(out_v, [lanes + (out0 + row)], acc, mask=lane0)


@functools.partial(
    pl.kernel,
    mesh=plsc.VectorSubcoreMesh(core_axis_name="c", subcore_axis_name="s"),
    out_type=jax.ShapeDtypeStruct((BATCH,), jnp.float32),
    scratch_types=[
        pltpu.VMEM((BPW,), jnp.int32),
        pltpu.VMEM((BPW,), jnp.int32),
        pltpu.VMEM((NBUF, CH, DIM), jnp.float32),
        pltpu.VMEM((NBUF, CH, DIM), jnp.float32),
        pltpu.VMEM((BPW,), jnp.float32),
    ] + [pltpu.SemaphoreType.DMA] * (NBUF + 1),
    compiler_params=pltpu.CompilerParams(needs_layout_passes=False),
)
def _w2v_kernel(center_hbm, context_hbm, w_in_hbm, w_out_hbm, out_hbm,
                cidx_v, xidx_v, a_v, b_v, out_v, *all_sems):
    wid = lax.axis_index("s") * NUM_CORES + lax.axis_index("c")
    base = wid * BPW

    sems = all_sems[:NBUF]
    sem2 = all_sems[NBUF]

    chunks = [(0, 32), (32, 32), (64, 64)]
    off = CH
    while off < BPW:
        chunks.append((off, CH))
        off += CH

    first = chunks[0][1] + chunks[1][1]
    hc = pltpu.async_copy(
        center_hbm.at[pl.ds(base, first)], cidx_v.at[pl.ds(0, first)], sem2)
    hx = pltpu.async_copy(
        context_hbm.at[pl.ds(base, first)], xidx_v.at[pl.ds(0, first)], sem2)
    hc.wait()
    hx.wait()

    def start(i):
        off, n = chunks[i]
        slot = i % NBUF
        ha = pltpu.async_copy(
            w_in_hbm.at[cidx_v.at[pl.ds(off, n)]],
            a_v.at[slot, pl.ds(0, n)], sems[slot])
        hb = pltpu.async_copy(
            w_out_hbm.at[xidx_v.at[pl.ds(off, n)]],
            b_v.at[slot, pl.ds(0, n)], sems[slot])
        return ha, hb

    pending = [start(j) for j in range(NBUF - 1)]

    rest = BPW - first
    hc = pltpu.async_copy(
        center_hbm.at[pl.ds(base + first, rest)],
        cidx_v.at[pl.ds(first, rest)], sem2)
    hx = pltpu.async_copy(
        context_hbm.at[pl.ds(base + first, rest)],
        xidx_v.at[pl.ds(first, rest)], sem2)
    hc.wait()
    hx.wait()

    hout = None
    for i, (off, n) in enumerate(chunks):
        if i + NBUF - 1 < len(chunks):
            pending.append(start(i + NBUF - 1))
        ha, hb = pending.pop(0)
        ha.wait()
        hb.wait()
        slot = i % NBUF

        _dot_chunk(a_v, b_v, out_v, slot, off, n)

        if hout is not None:
            hout.wait()
        hout = pltpu.async_copy(
            out_v.at[pl.ds(off, n)],
            out_hbm.at[pl.ds(base + off, n)], sem2)

    hout.wait()


def kernel(center, context, W_in, W_out):
    return _w2v_kernel(center, context, W_in, W_out)


# speedup vs baseline: 1.0205x; 1.0205x over previous
import functools

import jax
import jax.numpy as jnp
from jax import lax
from jax.experimental import pallas as pl
from jax.experimental.pallas import tpu as pltpu
from jax.experimental.pallas import tpu_sc as plsc

VOCAB = 100000
DIM = 128
BATCH = 16384

NUM_CORES = 2
NUM_SUBCORES = 16
NW = NUM_CORES * NUM_SUBCORES
BPW = BATCH // NW
CH = 128
NCH = BPW // CH
NBUF = 3
LANES = 16


def _lane_permute(x, idx):
    dnums = lax.GatherDimensionNumbers(
        offset_dims=(), collapsed_slice_dims=(0,), start_index_map=(0,))
    return lax.gather(
        x, idx[:, None], dnums, slice_sizes=(1,),
        mode=lax.GatherScatterMode.PROMISE_IN_BOUNDS)


def _dot_chunk(a_v, b_v, out_v, slot, out0, n):
    lanes = lax.broadcasted_iota(jnp.int32, (LANES,), 0)
    perms = [lanes ^ step for step in (8, 4, 2, 1)]
    lane0 = lanes == 0

    @plsc.parallel_loop(0, n)
    def _(row):
        acc = None
        for k in range(DIM // LANES):
            av = a_v[slot, row, pl.ds(k * LANES, LANES)]
            bv = b_v[slot, row, pl.ds(k * LANES, LANES)]
            prod = av * bv
            acc = prod if acc is None else acc + prod
        for perm in perms:
            acc = acc + _lane_permute(acc, perm)
        plsc.store_scatter(out_v, [lanes + (out0 + row)], acc, mask=lane0)


@functools.partial(
    pl.kernel,
    mesh=plsc.VectorSubcoreMesh(core_axis_name="c", subcore_axis_name="s"),
    out_type=jax.ShapeDtypeStruct((BATCH,), jnp.float32),
    scratch_types=[
        pltpu.VMEM((BPW,), jnp.int32),
        pltpu.VMEM((BPW,), jnp.int32),
        pltpu.VMEM((NBUF, CH, DIM), jnp.float32),
        pltpu.VMEM((NBUF, CH, DIM), jnp.float32),
        pltpu.VMEM((BPW,), jnp.float32),
    ] + [pltpu.SemaphoreType.DMA] * (NBUF + 1),
    compiler_params=pltpu.CompilerParams(needs_layout_passes=False),
)
def _w2v_kernel(center_hbm, context_hbm, w_in_hbm, w_out_hbm, out_hbm,
                cidx_v, xidx_v, a_v, b_v, out_v, *all_sems):
    wid = lax.axis_index("s") * NUM_CORES + lax.axis_index("c")
    base = wid * BPW

    sems = all_sems[:NBUF]
    sem2 = all_sems[NBUF]

    chunks = [(0, CH // 2), (CH // 2, CH // 2)]
    off = CH
    while off < BPW:
        chunks.append((off, CH))
        off += CH

    first = chunks[0][1] + chunks[1][1]
    hc = pltpu.async_copy(
        center_hbm.at[pl.ds(base, first)], cidx_v.at[pl.ds(0, first)], sem2)
    hx = pltpu.async_copy(
        context_hbm.at[pl.ds(base, first)], xidx_v.at[pl.ds(0, first)], sem2)
    hc.wait()
    hx.wait()

    def start(i):
        off, n = chunks[i]
        slot = i % NBUF
        ha = pltpu.async_copy(
            w_in_hbm.at[cidx_v.at[pl.ds(off, n)]],
            a_v.at[slot, pl.ds(0, n)], sems[slot])
        hb = pltpu.async_copy(
            w_out_hbm.at[xidx_v.at[pl.ds(off, n)]],
            b_v.at[slot, pl.ds(0, n)], sems[slot])
        return ha, hb

    pending = [start(j) for j in range(NBUF - 1)]

    rest = BPW - first
    hc = pltpu.async_copy(
        center_hbm.at[pl.ds(base + first, rest)],
        cidx_v.at[pl.ds(first, rest)], sem2)
    hx = pltpu.async_copy(
        context_hbm.at[pl.ds(base + first, rest)],
        xidx_v.at[pl.ds(first, rest)], sem2)
    hc.wait()
    hx.wait()

    hout = None
    for i, (off, n) in enumerate(chunks):
        if i + NBUF - 1 < len(chunks):
            pending.append(start(i + NBUF - 1))
        ha, hb = pending.pop(0)
        ha.wait()
        hb.wait()
        slot = i % NBUF

        _dot_chunk(a_v, b_v, out_v, slot, off, n)

        if hout is not None:
            hout.wait()
        hout = pltpu.async_copy(
            out_v.at[pl.ds(off, n)],
            out_hbm.at[pl.ds(base + off, n)], sem2)

    hout.wait()


def kernel(center, context, W_in, W_out):
    return _w2v_kernel(center, context, W_in, W_out)


# speedup vs baseline: 1.0560x; 1.0348x over previous
import functools

import jax
import jax.numpy as jnp
from jax import lax
from jax.experimental import pallas as pl
from jax.experimental.pallas import tpu as pltpu
from jax.experimental.pallas import tpu_sc as plsc

VOCAB = 100000
DIM = 128
BATCH = 16384

NUM_CORES = 2
NUM_SUBCORES = 16
NW = NUM_CORES * NUM_SUBCORES
BPW = BATCH // NW
CH = 128
NCH = BPW // CH
NBUF = 3
LANES = 16


def _lane_permute(x, idx):
    dnums = lax.GatherDimensionNumbers(
        offset_dims=(), collapsed_slice_dims=(0,), start_index_map=(0,))
    return lax.gather(
        x, idx[:, None], dnums, slice_sizes=(1,),
        mode=lax.GatherScatterMode.PROMISE_IN_BOUNDS)


def _dot_chunk(a_v, b_v, out_v, slot, out0, n):
    lanes = lax.broadcasted_iota(jnp.int32, (LANES,), 0)
    perms = [lanes ^ step for step in (8, 4, 2, 1)]
    lane0 = lanes == 0

    @plsc.parallel_loop(0, n)
    def _(row):
        acc = None
        for k in range(DIM // LANES):
            av = a_v[slot, row, pl.ds(k * LANES, LANES)]
            bv = b_v[slot, row, pl.ds(k * LANES, LANES)]
            prod = av * bv
            acc = prod if acc is None else acc + prod
        for perm in perms:
            acc = acc + _lane_permute(acc, perm)
        plsc.store_scatter(out_v, [lanes + (out0 + row)], acc, mask=lane0)


@functools.partial(
    pl.kernel,
    mesh=plsc.VectorSubcoreMesh(core_axis_name="c", subcore_axis_name="s"),
    out_type=jax.ShapeDtypeStruct((BATCH,), jnp.float32),
    scratch_types=[
        pltpu.VMEM((BPW,), jnp.int32),
        pltpu.VMEM((BPW,), jnp.int32),
        pltpu.VMEM((NBUF, CH, DIM), jnp.float32),
        pltpu.VMEM((NBUF, CH, DIM), jnp.float32),
        pltpu.VMEM((BPW,), jnp.float32),
    ] + [pltpu.SemaphoreType.DMA] * (NBUF + 1),
    compiler_params=pltpu.CompilerParams(needs_layout_passes=False),
)
def _w2v_kernel(center_hbm, context_hbm, w_in_hbm, w_out_hbm, out_hbm,
                cidx_v, xidx_v, a_v, b_v, out_v, *all_sems):
    wid = lax.axis_index("s") * NUM_CORES + lax.axis_index("c")
    base = wid * BPW

    sems = all_sems[:NBUF]
    sem2 = all_sems[NBUF]

    chunks = [(0, CH // 2), (CH // 2, CH // 2)]
    off = CH
    while off < BPW:
        chunks.append((off, CH))
        off += CH

    first = chunks[0][1]
    hc = pltpu.async_copy(
        center_hbm.at[pl.ds(base, first)], cidx_v.at[pl.ds(0, first)], sem2)
    hx = pltpu.async_copy(
        context_hbm.at[pl.ds(base, first)], xidx_v.at[pl.ds(0, first)], sem2)
    hc.wait()
    hx.wait()

    def start(i):
        off, n = chunks[i]
        slot = i % NBUF
        ha = pltpu.async_copy(
            w_in_hbm.at[cidx_v.at[pl.ds(off, n)]],
            a_v.at[slot, pl.ds(0, n)], sems[slot])
        hb = pltpu.async_copy(
            w_out_hbm.at[xidx_v.at[pl.ds(off, n)]],
            b_v.at[slot, pl.ds(0, n)], sems[slot])
        return ha, hb

    pending = [start(0)]

    rest = BPW - first
    hc = pltpu.async_copy(
        center_hbm.at[pl.ds(base + first, rest)],
        cidx_v.at[pl.ds(first, rest)], sem2)
    hx = pltpu.async_copy(
        context_hbm.at[pl.ds(base + first, rest)],
        xidx_v.at[pl.ds(first, rest)], sem2)
    hc.wait()
    hx.wait()
    for j in range(1, NBUF - 1):
        pending.append(start(j))

    hout = None
    for i, (off, n) in enumerate(chunks):
        if i + NBUF - 1 < len(chunks):
            pending.append(start(i + NBUF - 1))
        ha, hb = pending.pop(0)
        ha.wait()
        hb.wait()
        slot = i % NBUF

        _dot_chunk(a_v, b_v, out_v, slot, off, n)

        if hout is not None:
            hout.wait()
        hout = pltpu.async_copy(
            out_v.at[pl.ds(off, n)],
            out_hbm.at[pl.ds(base + off, n)], sem2)

    hout.wait()


def kernel(center, context, W_in, W_out):
    return _w2v_kernel(center, context, W_in, W_out)
